# baseline (device time: 114452 ns/iter reference)
import jax
import jax.numpy as jnp
from jax import lax
from jax.experimental import pallas as pl
from jax.experimental.pallas import tpu as pltpu

N_DEV = 8
B = 2
SQ = 128
D = 512
SKV_LOC = 128
HQ_LOC = 4
DH = 64


def kernel(x, Wq, K_ext, V_ext, Wo):
    def body(x_ref, wq_ref, k_ref, v_ref, wo_ref, out_ref,
             k_send, v_send, k_all, v_all, mine, out_parts,
             send_sems, recv_sems, local_sems):
        my = lax.axis_index("i")

        barrier_sem = pltpu.get_barrier_semaphore()
        for k in range(1, N_DEV):
            pl.semaphore_signal(
                barrier_sem, inc=1,
                device_id=((my + k) % N_DEV,),
                device_id_type=pl.DeviceIdType.MESH,
            )
        pl.semaphore_wait(barrier_sem, N_DEV - 1)

        for j in range(N_DEV):
            k_send[j] = k_ref[:, :, HQ_LOC * j:HQ_LOC * (j + 1), :]
            v_send[j] = v_ref[:, :, HQ_LOC * j:HQ_LOC * (j + 1), :]

        cp_k = pltpu.make_async_copy(k_send.at[my], k_all.at[my], local_sems.at[0])
        cp_v = pltpu.make_async_copy(v_send.at[my], v_all.at[my], local_sems.at[1])
        cp_k.start()
        cp_v.start()

        kv_rdmas = []
        for k in range(1, N_DEV):
            j = (my + k) % N_DEV
            rk = pltpu.make_async_remote_copy(
                src_ref=k_send.at[j], dst_ref=k_all.at[my],
                send_sem=send_sems.at[0, k - 1], recv_sem=recv_sems.at[0, k - 1],
                device_id=(j,), device_id_type=pl.DeviceIdType.MESH,
            )
            rv = pltpu.make_async_remote_copy(
                src_ref=v_send.at[j], dst_ref=v_all.at[my],
                send_sem=send_sems.at[1, k - 1], recv_sem=recv_sems.at[1, k - 1],
                device_id=(j,), device_id_type=pl.DeviceIdType.MESH,
            )
            rk.start()
            rv.start()
            kv_rdmas.append((rk, rv))

        x2d = x_ref[...].reshape(B * SQ, D)
        q2d = jnp.dot(x2d, wq_ref[...], preferred_element_type=jnp.float32)

        cp_k.wait()
        cp_v.wait()
        for rk, rv in kv_rdmas:
            rk.wait_recv()
            rv.wait_recv()

        kv_full = k_all[...]
        vv_full = v_all[...]
        skv = N_DEV * SKV_LOC
        qi = lax.broadcasted_iota(jnp.int32, (SQ, skv), 0)
        ki = lax.broadcasted_iota(jnp.int32, (SQ, skv), 1)
        mask = (jnp.abs(qi - ki) <= 128) | (ki < 32) | (qi < 32)

        ctx_rows = []
        for b in range(B):
            heads = []
            for h in range(HQ_LOC):
                qbh = q2d[b * SQ:(b + 1) * SQ, h * DH:(h + 1) * DH]
                kbh = kv_full[:, b, :, h, :].reshape(skv, DH)
                vbh = vv_full[:, b, :, h, :].reshape(skv, DH)
                s = lax.dot_general(
                    qbh, kbh, (((1,), (1,)), ((), ())),
                    preferred_element_type=jnp.float32,
                ) * 0.125
                s = jnp.where(mask, s, -1e9)
                m = jnp.max(s, axis=1, keepdims=True)
                w = jnp.exp(s - m)
                w = w / jnp.sum(w, axis=1, keepdims=True)
                heads.append(lax.dot_general(
                    w, vbh, (((1,), (0,)), ((), ())),
                    preferred_element_type=jnp.float32,
                ))
            ctx_rows.append(jnp.concatenate(heads, axis=1))
        ctx2d = jnp.concatenate(ctx_rows, axis=0)

        mine[...] = jnp.dot(ctx2d, wo_ref[...], preferred_element_type=jnp.float32)

        cp_m = pltpu.make_async_copy(mine, out_parts.at[my], local_sems.at[2])
        cp_m.start()
        p_rdmas = []
        for k in range(1, N_DEV):
            j = (my + k) % N_DEV
            r = pltpu.make_async_remote_copy(
                src_ref=mine, dst_ref=out_parts.at[my],
                send_sem=send_sems.at[2, k - 1], recv_sem=recv_sems.at[2, k - 1],
                device_id=(j,), device_id_type=pl.DeviceIdType.MESH,
            )
            r.start()
            p_rdmas.append(r)

        for rk, rv in kv_rdmas:
            rk.wait_send()
            rv.wait_send()
        cp_m.wait()
        for r in p_rdmas:
            r.wait_recv()

        total = jnp.sum(out_parts[...], axis=0)
        out_ref[...] = total.reshape(B, SQ, D)

        for r in p_rdmas:
            r.wait_send()

    return pl.pallas_call(
        body,
        out_shape=jax.ShapeDtypeStruct((B, SQ, D), jnp.float32),
        in_specs=[pl.BlockSpec(memory_space=pltpu.VMEM)] * 5,
        out_specs=pl.BlockSpec(memory_space=pltpu.VMEM),
        scratch_shapes=[
            pltpu.VMEM((N_DEV, B, SKV_LOC, HQ_LOC, DH), jnp.float32),
            pltpu.VMEM((N_DEV, B, SKV_LOC, HQ_LOC, DH), jnp.float32),
            pltpu.VMEM((N_DEV, B, SKV_LOC, HQ_LOC, DH), jnp.float32),
            pltpu.VMEM((N_DEV, B, SKV_LOC, HQ_LOC, DH), jnp.float32),
            pltpu.VMEM((B * SQ, D), jnp.float32),
            pltpu.VMEM((N_DEV, B * SQ, D), jnp.float32),
            pltpu.SemaphoreType.DMA((3, N_DEV - 1)),
            pltpu.SemaphoreType.DMA((3, N_DEV - 1)),
            pltpu.SemaphoreType.DMA((3,)),
        ],
        compiler_params=pltpu.CompilerParams(collective_id=0),
    )(x, Wq, K_ext, V_ext, Wo)


# device time: 112244 ns/iter; 1.0197x vs baseline; 1.0197x over previous
import jax
import jax.numpy as jnp
from jax import lax
from jax.experimental import pallas as pl
from jax.experimental.pallas import tpu as pltpu

N_DEV = 8
B = 2
SQ = 128
D = 512
SKV_LOC = 128
HQ_LOC = 4
DH = 64


def kernel(x, Wq, K_ext, V_ext, Wo):
    def _finish(my, mine, out_parts, out_ref,
                send_sems, recv_sems, local_sems, kv_rdmas):
        cp_m = pltpu.make_async_copy(mine, out_parts.at[my], local_sems.at[2])
        cp_m.start()
        p_rdmas = []
        for k in range(1, N_DEV):
            j = (my + k) % N_DEV
            r = pltpu.make_async_remote_copy(
                src_ref=mine, dst_ref=out_parts.at[my],
                send_sem=send_sems.at[2, k - 1], recv_sem=recv_sems.at[2, k - 1],
                device_id=(j,), device_id_type=pl.DeviceIdType.MESH,
            )
            r.start()
            p_rdmas.append(r)

        for rk, rv in kv_rdmas:
            rk.wait_send()
            rv.wait_send()
        cp_m.wait()
        for r in p_rdmas:
            r.wait_recv()

        total = jnp.sum(out_parts[...], axis=0)
        out_ref[...] = total.reshape(B, SQ, D)

        for r in p_rdmas:
            r.wait_send()

    def body(x_ref, wq_ref, k_ref, v_ref, wo_ref, out_ref,
             k_send, v_send, k_all, v_all, mine, out_parts,
             send_sems, recv_sems, local_sems):
        my = lax.axis_index("i")

        barrier_sem = pltpu.get_barrier_semaphore()
        for k in range(1, N_DEV):
            pl.semaphore_signal(
                barrier_sem, inc=1,
                device_id=((my + k) % N_DEV,),
                device_id_type=pl.DeviceIdType.MESH,
            )
        pl.semaphore_wait(barrier_sem, N_DEV - 1)

        for j in range(N_DEV):
            k_send[j] = k_ref[:, :, HQ_LOC * j:HQ_LOC * (j + 1), :]
            v_send[j] = v_ref[:, :, HQ_LOC * j:HQ_LOC * (j + 1), :]

        cp_k = pltpu.make_async_copy(k_send.at[my], k_all.at[my], local_sems.at[0])
        cp_v = pltpu.make_async_copy(v_send.at[my], v_all.at[my], local_sems.at[1])
        cp_k.start()
        cp_v.start()

        kv_rdmas = []
        for k in range(1, N_DEV):
            j = (my + k) % N_DEV
            rk = pltpu.make_async_remote_copy(
                src_ref=k_send.at[j], dst_ref=k_all.at[my],
                send_sem=send_sems.at[0, k - 1], recv_sem=recv_sems.at[0, k - 1],
                device_id=(j,), device_id_type=pl.DeviceIdType.MESH,
            )
            rv = pltpu.make_async_remote_copy(
                src_ref=v_send.at[j], dst_ref=v_all.at[my],
                send_sem=send_sems.at[1, k - 1], recv_sem=recv_sems.at[1, k - 1],
                device_id=(j,), device_id_type=pl.DeviceIdType.MESH,
            )
            rk.start()
            rv.start()
            kv_rdmas.append((rk, rv))

        x2d = x_ref[...].reshape(B * SQ, D)
        q2d = jnp.dot(x2d, wq_ref[...], preferred_element_type=jnp.float32)

        cp_k.wait()
        cp_v.wait()
        for rk, rv in kv_rdmas:
            rk.wait_recv()
            rv.wait_recv()

        PROBE_NO_COMPUTE = True
        if PROBE_NO_COMPUTE:
            ctx2d = q2d + k_all[0, 0, 0, 0, 0] + v_all[0, 0, 0, 0, 0]
            mine[...] = jnp.dot(ctx2d, wo_ref[...],
                                preferred_element_type=jnp.float32)
            _finish(my, mine, out_parts, out_ref,
                    send_sems, recv_sems, local_sems, kv_rdmas)
            return

        kv_full = k_all[...]
        vv_full = v_all[...]
        skv = N_DEV * SKV_LOC
        qi = lax.broadcasted_iota(jnp.int32, (SQ, skv), 0)
        ki = lax.broadcasted_iota(jnp.int32, (SQ, skv), 1)
        mask = (jnp.abs(qi - ki) <= 128) | (ki < 32) | (qi < 32)

        ctx_rows = []
        for b in range(B):
            heads = []
            for h in range(HQ_LOC):
                qbh = q2d[b * SQ:(b + 1) * SQ, h * DH:(h + 1) * DH]
                kbh = kv_full[:, b, :, h, :].reshape(skv, DH)
                vbh = vv_full[:, b, :, h, :].reshape(skv, DH)
                s = lax.dot_general(
                    qbh, kbh, (((1,), (1,)), ((), ())),
                    preferred_element_type=jnp.float32,
                ) * 0.125
                s = jnp.where(mask, s, -1e9)
                m = jnp.max(s, axis=1, keepdims=True)
                w = jnp.exp(s - m)
                w = w / jnp.sum(w, axis=1, keepdims=True)
                heads.append(lax.dot_general(
                    w, vbh, (((1,), (0,)), ((), ())),
                    preferred_element_type=jnp.float32,
                ))
            ctx_rows.append(jnp.concatenate(heads, axis=1))
        ctx2d = jnp.concatenate(ctx_rows, axis=0)

        mine[...] = jnp.dot(ctx2d, wo_ref[...], preferred_element_type=jnp.float32)
        _finish(my, mine, out_parts, out_ref,
                send_sems, recv_sems, local_sems, kv_rdmas)

    return pl.pallas_call(
        body,
        out_shape=jax.ShapeDtypeStruct((B, SQ, D), jnp.float32),
        in_specs=[pl.BlockSpec(memory_space=pltpu.VMEM)] * 5,
        out_specs=pl.BlockSpec(memory_space=pltpu.VMEM),
        scratch_shapes=[
            pltpu.VMEM((N_DEV, B, SKV_LOC, HQ_LOC, DH), jnp.float32),
            pltpu.VMEM((N_DEV, B, SKV_LOC, HQ_LOC, DH), jnp.float32),
            pltpu.VMEM((N_DEV, B, SKV_LOC, HQ_LOC, DH), jnp.float32),
            pltpu.VMEM((N_DEV, B, SKV_LOC, HQ_LOC, DH), jnp.float32),
            pltpu.VMEM((B * SQ, D), jnp.float32),
            pltpu.VMEM((N_DEV, B * SQ, D), jnp.float32),
            pltpu.SemaphoreType.DMA((3, N_DEV - 1)),
            pltpu.SemaphoreType.DMA((3, N_DEV - 1)),
            pltpu.SemaphoreType.DMA((3,)),
        ],
        compiler_params=pltpu.CompilerParams(collective_id=0),
    )(x, Wq, K_ext, V_ext, Wo)


# device time: 70006 ns/iter; 1.6349x vs baseline; 1.6033x over previous
import jax
import jax.numpy as jnp
from jax import lax
from jax.experimental import pallas as pl
from jax.experimental.pallas import tpu as pltpu

N_DEV = 8
B = 2
SQ = 128
D = 512
SKV_LOC = 128
HQ_LOC = 4
DH = 64
DBLK = D // N_DEV


def kernel(x, Wq, K_ext, V_ext, Wo):
    def body(x_ref, wq_ref, k_ref, v_ref, wo_ref, out_ref,
             k_send, v_send, k_all, v_all, mine_bf, rs_buf, ag_send, ag_buf,
             send_sems, recv_sems, local_sems):
        my = lax.axis_index("i")

        barrier_sem = pltpu.get_barrier_semaphore()
        for k in range(1, N_DEV):
            pl.semaphore_signal(
                barrier_sem, inc=1,
                device_id=((my + k) % N_DEV,),
                device_id_type=pl.DeviceIdType.MESH,
            )
        pl.semaphore_wait(barrier_sem, N_DEV - 1)

        for j in range(N_DEV):
            k_send[j] = k_ref[:, :, HQ_LOC * j:HQ_LOC * (j + 1), :].astype(
                jnp.bfloat16)
            v_send[j] = v_ref[:, :, HQ_LOC * j:HQ_LOC * (j + 1), :].astype(
                jnp.bfloat16)

        cp_k = pltpu.make_async_copy(k_send.at[my], k_all.at[my], local_sems.at[0])
        cp_v = pltpu.make_async_copy(v_send.at[my], v_all.at[my], local_sems.at[1])
        cp_k.start()
        cp_v.start()

        kv_rdmas = []
        for k in range(1, N_DEV):
            j = (my + k) % N_DEV
            rk = pltpu.make_async_remote_copy(
                src_ref=k_send.at[j], dst_ref=k_all.at[my],
                send_sem=send_sems.at[0, k - 1], recv_sem=recv_sems.at[0, k - 1],
                device_id=(j,), device_id_type=pl.DeviceIdType.MESH,
            )
            rv = pltpu.make_async_remote_copy(
                src_ref=v_send.at[j], dst_ref=v_all.at[my],
                send_sem=send_sems.at[1, k - 1], recv_sem=recv_sems.at[1, k - 1],
                device_id=(j,), device_id_type=pl.DeviceIdType.MESH,
            )
            rk.start()
            rv.start()
            kv_rdmas.append((rk, rv))

        x2d = x_ref[...].reshape(B * SQ, D)
        q2d = jnp.dot(x2d, wq_ref[...],
                      preferred_element_type=jnp.float32).astype(jnp.bfloat16)

        cp_k.wait()
        cp_v.wait()
        for rk, rv in kv_rdmas:
            rk.wait_recv()
            rv.wait_recv()

        kv_full = k_all[...]
        vv_full = v_all[...]
        skv = N_DEV * SKV_LOC
        qi = lax.broadcasted_iota(jnp.int32, (SQ, skv), 0)
        ki = lax.broadcasted_iota(jnp.int32, (SQ, skv), 1)
        mask = (jnp.abs(qi - ki) <= 128) | (ki < 32) | (qi < 32)

        ctx_rows = []
        for b in range(B):
            heads = []
            for h in range(HQ_LOC):
                qbh = q2d[b * SQ:(b + 1) * SQ, h * DH:(h + 1) * DH]
                kbh = kv_full[:, b, :, h, :].reshape(skv, DH)
                vbh = vv_full[:, b, :, h, :].reshape(skv, DH)
                s = lax.dot_general(
                    qbh, kbh, (((1,), (1,)), ((), ())),
                    preferred_element_type=jnp.float32,
                ) * 0.125
                s = jnp.where(mask, s, -1e9)
                m = jnp.max(s, axis=1, keepdims=True)
                w = jnp.exp(s - m)
                w = (w / jnp.sum(w, axis=1, keepdims=True)).astype(jnp.bfloat16)
                heads.append(lax.dot_general(
                    w, vbh, (((1,), (0,)), ((), ())),
                    preferred_element_type=jnp.float32,
                ))
            ctx_rows.append(jnp.concatenate(heads, axis=1))
        ctx2d = jnp.concatenate(ctx_rows, axis=0)

        partial = jnp.dot(ctx2d, wo_ref[...],
                          preferred_element_type=jnp.float32)
        for j in range(N_DEV):
            mine_bf[j] = partial[:, DBLK * j:DBLK * (j + 1)].astype(jnp.bfloat16)

        cp_rs = pltpu.make_async_copy(mine_bf.at[my], rs_buf.at[my],
                                      local_sems.at[2])
        cp_rs.start()
        rs_rdmas = []
        for k in range(1, N_DEV):
            j = (my + k) % N_DEV
            r = pltpu.make_async_remote_copy(
                src_ref=mine_bf.at[j], dst_ref=rs_buf.at[my],
                send_sem=send_sems.at[2, k - 1], recv_sem=recv_sems.at[2, k - 1],
                device_id=(j,), device_id_type=pl.DeviceIdType.MESH,
            )
            r.start()
            rs_rdmas.append(r)

        for rk, rv in kv_rdmas:
            rk.wait_send()
            rv.wait_send()
        cp_rs.wait()
        for r in rs_rdmas:
            r.wait_recv()

        blk = jnp.sum(rs_buf[...].astype(jnp.float32), axis=0)
        ag_send[...] = blk.astype(jnp.bfloat16)

        cp_ag = pltpu.make_async_copy(ag_send, ag_buf.at[my], local_sems.at[3])
        cp_ag.start()
        ag_rdmas = []
        for k in range(1, N_DEV):
            j = (my + k) % N_DEV
            r = pltpu.make_async_remote_copy(
                src_ref=ag_send, dst_ref=ag_buf.at[my],
                send_sem=send_sems.at[3, k - 1], recv_sem=recv_sems.at[3, k - 1],
                device_id=(j,), device_id_type=pl.DeviceIdType.MESH,
            )
            r.start()
            ag_rdmas.append(r)

        for r in rs_rdmas:
            r.wait_send()
        cp_ag.wait()
        for r in ag_rdmas:
            r.wait_recv()

        out2d = jnp.concatenate(
            [ag_buf[j].astype(jnp.float32) for j in range(N_DEV)], axis=1)
        out_ref[...] = out2d.reshape(B, SQ, D)

        for r in ag_rdmas:
            r.wait_send()

    return pl.pallas_call(
        body,
        out_shape=jax.ShapeDtypeStruct((B, SQ, D), jnp.float32),
        in_specs=[pl.BlockSpec(memory_space=pltpu.VMEM)] * 5,
        out_specs=pl.BlockSpec(memory_space=pltpu.VMEM),
        scratch_shapes=[
            pltpu.VMEM((N_DEV, B, SKV_LOC, HQ_LOC, DH), jnp.bfloat16),
            pltpu.VMEM((N_DEV, B, SKV_LOC, HQ_LOC, DH), jnp.bfloat16),
            pltpu.VMEM((N_DEV, B, SKV_LOC, HQ_LOC, DH), jnp.bfloat16),
            pltpu.VMEM((N_DEV, B, SKV_LOC, HQ_LOC, DH), jnp.bfloat16),
            pltpu.VMEM((N_DEV, B * SQ, DBLK), jnp.bfloat16),
            pltpu.VMEM((N_DEV, B * SQ, DBLK), jnp.bfloat16),
            pltpu.VMEM((B * SQ, DBLK), jnp.bfloat16),
            pltpu.VMEM((N_DEV, B * SQ, DBLK), jnp.bfloat16),
            pltpu.SemaphoreType.DMA((4, N_DEV - 1)),
            pltpu.SemaphoreType.DMA((4, N_DEV - 1)),
            pltpu.SemaphoreType.DMA((4,)),
        ],
        compiler_params=pltpu.CompilerParams(collective_id=0),
    )(x, Wq, K_ext, V_ext, Wo)
